# Initial kernel scaffold; baseline (speedup 1.0000x reference)
import jax
import jax.numpy as jnp
from jax.experimental import pallas as pl


def kernel(x, edge_index, decode_index, W1l, b1l, W1r, W2l, b2l, W2r):
    raise NotImplementedError("write your pallas kernel here")


# speedup vs baseline: 10.1640x; 10.1640x over previous
import functools

import jax
import jax.numpy as jnp
from jax import lax
from jax.experimental import pallas as pl
from jax.experimental.pallas import tpu as pltpu
from jax.experimental.pallas import tpu_sc as plsc

NC, NS, L = 2, 16, 16
NW = NC * NS

N = 10000
E = 320000
D = 128
H = 16
Q = 100000

CHUNK = 128
E_PAD = 327680
E_CHUNKS_PER_TILE = E_PAD // (NW * CHUNK)
NPAD = 10240
ROWS_PER_TILE = NPAD // NS
Q_PAD = 102400
Q_CHUNKS_PER_TILE = Q_PAD // (NW * CHUNK)
Q_PER_TILE = Q_PAD // NW

_f32 = jnp.float32
_i32 = jnp.int32


def _mesh():
    return plsc.VectorSubcoreMesh(
        core_axis_name="c", subcore_axis_name="s",
        num_cores=NC, num_subcores=NS)


def _make_seg_pass(with_count):
    out_type = [jax.ShapeDtypeStruct((NC, NPAD, H), _f32)]
    scratch = [
        pltpu.VMEM_SHARED((NPAD, H), _f32),
        pltpu.VMEM((E_CHUNKS_PER_TILE, CHUNK), _i32),
        pltpu.VMEM((E_CHUNKS_PER_TILE, CHUNK), _i32),
        pltpu.VMEM((CHUNK, H), _f32),
        pltpu.SemaphoreType.DMA,
    ]
    if with_count:
        out_type.append(jax.ShapeDtypeStruct((NC, NPAD), _f32))
        scratch += [
            pltpu.VMEM_SHARED((NPAD,), _f32),
            pltpu.VMEM((CHUNK,), _f32),
        ]

    def body(table, src2d, dst2d, z16, z1, ones, *rest):
        if with_count:
            (agg_out, cnt_out, agg_sh, src_v, dst_v, rows_v, sem,
             cnt_sh, ones_v) = rest
        else:
            agg_out, agg_sh, src_v, dst_v, rows_v, sem = rest
        c = lax.axis_index("c")
        s = lax.axis_index("s")
        t = c * NS + s
        nck = E_CHUNKS_PER_TILE
        pltpu.sync_copy(src2d.at[pl.ds(t * nck, nck)], src_v)
        pltpu.sync_copy(dst2d.at[pl.ds(t * nck, nck)], dst_v)
        rpt = ROWS_PER_TILE
        pltpu.sync_copy(z16.at[pl.ds(s * rpt, rpt)], agg_sh.at[pl.ds(s * rpt, rpt)])
        if with_count:
            pltpu.sync_copy(ones, ones_v)
            pltpu.sync_copy(z1.at[pl.ds(s * rpt, rpt)], cnt_sh.at[pl.ds(s * rpt, rpt)])
        plsc.subcore_barrier()

        def chunk(j, carry):
            pltpu.async_copy(table.at[src_v.at[j]], rows_v, sem).wait()
            pltpu.sync_copy(rows_v, agg_sh.at[dst_v.at[j]], add=True)
            if with_count:
                pltpu.sync_copy(ones_v, cnt_sh.at[dst_v.at[j]], add=True)
            return carry

        lax.fori_loop(0, nck, chunk, 0)
        plsc.subcore_barrier()
        pltpu.sync_copy(agg_sh.at[pl.ds(s * rpt, rpt)],
                        agg_out.at[c, pl.ds(s * rpt, rpt)])
        if with_count:
            pltpu.sync_copy(cnt_sh.at[pl.ds(s * rpt, rpt)],
                            cnt_out.at[c, pl.ds(s * rpt, rpt)])

    return pl.kernel(body, out_type=tuple(out_type), mesh=_mesh(),
                     scratch_types=scratch,
                     compiler_params=pltpu.CompilerParams(
                         use_tc_tiling_on_sc=False))


def _decode_body(h2, ia2d, ib2d, out, ia_v, ib_v, ra_v, rb_v, res_v, sem):
    c = lax.axis_index("c")
    s = lax.axis_index("s")
    t = c * NS + s
    nck = Q_CHUNKS_PER_TILE
    pltpu.sync_copy(ia2d.at[pl.ds(t * nck, nck)], ia_v)
    pltpu.sync_copy(ib2d.at[pl.ds(t * nck, nck)], ib_v)
    lane = lax.iota(_i32, 16)

    def chunk(j, carry):
        ca = pltpu.async_copy(h2.at[ia_v.at[j]], ra_v, sem)
        cb = pltpu.async_copy(h2.at[ib_v.at[j]], rb_v, sem)
        ca.wait()
        cb.wait()
        for g in range(CHUNK // 16):
            ridx = lane + g * 16
            acc = jnp.zeros((16,), _f32)
            for col in range(H):
                cidx = jnp.full((16,), col, _i32)
                a = plsc.load_gather(ra_v, [ridx, cidx])
                b = plsc.load_gather(rb_v, [ridx, cidx])
                acc = acc + a * b
            sig = 1.0 / (1.0 + jnp.exp(-acc))
            res_v[pl.ds(j * CHUNK + g * 16, 16)] = sig
        return carry

    lax.fori_loop(0, nck, chunk, 0)
    pltpu.sync_copy(res_v, out.at[pl.ds(t * Q_PER_TILE, Q_PER_TILE)])


def _make_decode():
    return pl.kernel(
        _decode_body,
        out_type=jax.ShapeDtypeStruct((Q_PAD,), _f32),
        mesh=_mesh(),
        scratch_types=[
            pltpu.VMEM((Q_CHUNKS_PER_TILE, CHUNK), _i32),
            pltpu.VMEM((Q_CHUNKS_PER_TILE, CHUNK), _i32),
            pltpu.VMEM((CHUNK, H), _f32),
            pltpu.VMEM((CHUNK, H), _f32),
            pltpu.VMEM((Q_PER_TILE,), _f32),
            pltpu.SemaphoreType.DMA,
        ],
        compiler_params=pltpu.CompilerParams(use_tc_tiling_on_sc=False,
                                             needs_layout_passes=False))


_BN = 1000


def _mm_body(x_ref, w_ref, o_ref):
    o_ref[...] = jnp.dot(x_ref[...], w_ref[...], preferred_element_type=_f32)


def _project(x, wcat):
    k = x.shape[1]
    m = wcat.shape[1]
    return pl.pallas_call(
        _mm_body,
        grid=(N // _BN,),
        in_specs=[pl.BlockSpec((_BN, k), lambda i: (i, 0)),
                  pl.BlockSpec((k, m), lambda i: (0, 0))],
        out_specs=pl.BlockSpec((_BN, m), lambda i: (i, 0)),
        out_shape=jax.ShapeDtypeStruct((N, m), _f32),
    )(x, wcat)


def _comb1_body(a0, a1, c0, c1, xr, b, w, o_hlr, o_inv):
    cnt = c0[...] + c1[...]
    inv = 1.0 / jnp.maximum(cnt, 1.0)
    h1 = jnp.maximum((a0[...] + a1[...]) * inv + b[...] + xr[...], 0.0)
    o_hlr[...] = jnp.dot(h1, w[...], preferred_element_type=_f32)
    o_inv[...] = inv


def _combine1(a0, a1, c0, c1, xr, b1l2d, w2cat):
    return pl.pallas_call(
        _comb1_body,
        grid=(N // _BN,),
        in_specs=[pl.BlockSpec((_BN, H), lambda i: (i, 0)),
                  pl.BlockSpec((_BN, H), lambda i: (i, 0)),
                  pl.BlockSpec((_BN, 1), lambda i: (i, 0)),
                  pl.BlockSpec((_BN, 1), lambda i: (i, 0)),
                  pl.BlockSpec((_BN, H), lambda i: (i, 0)),
                  pl.BlockSpec((1, H), lambda i: (0, 0)),
                  pl.BlockSpec((H, 2 * H), lambda i: (0, 0))],
        out_specs=[pl.BlockSpec((_BN, 2 * H), lambda i: (i, 0)),
                   pl.BlockSpec((_BN, 1), lambda i: (i, 0))],
        out_shape=[jax.ShapeDtypeStruct((N, 2 * H), _f32),
                   jax.ShapeDtypeStruct((N, 1), _f32)],
    )(a0, a1, c0, c1, xr, b1l2d, w2cat)


def _comb2_body(a0, a1, inv, hr, b, o):
    o[...] = (a0[...] + a1[...]) * inv[...] + b[...] + hr[...]


def _combine2(a0, a1, inv, h1r, b2l2d):
    return pl.pallas_call(
        _comb2_body,
        grid=(N // _BN,),
        in_specs=[pl.BlockSpec((_BN, H), lambda i: (i, 0)),
                  pl.BlockSpec((_BN, H), lambda i: (i, 0)),
                  pl.BlockSpec((_BN, 1), lambda i: (i, 0)),
                  pl.BlockSpec((_BN, H), lambda i: (i, 0)),
                  pl.BlockSpec((1, H), lambda i: (0, 0))],
        out_specs=pl.BlockSpec((_BN, H), lambda i: (i, 0)),
        out_shape=jax.ShapeDtypeStruct((N, H), _f32),
    )(a0, a1, inv, h1r, b2l2d)


def kernel(x, edge_index, decode_index, W1l, b1l, W1r, W2l, b2l, W2r):
    src = edge_index[0]
    dst = edge_index[1]
    src2d = jnp.pad(src, (0, E_PAD - E)).reshape(E_PAD // CHUNK, CHUNK)
    dst2d = jnp.pad(dst, (0, E_PAD - E), constant_values=N).reshape(
        E_PAD // CHUNK, CHUNK)
    ia2d = jnp.pad(decode_index[0], (0, Q_PAD - Q)).reshape(Q_PAD // CHUNK, CHUNK)
    ib2d = jnp.pad(decode_index[1], (0, Q_PAD - Q)).reshape(Q_PAD // CHUNK, CHUNK)

    z16 = jnp.zeros((NPAD, H), _f32)
    z1 = jnp.zeros((NPAD,), _f32)
    ones = jnp.ones((CHUNK,), _f32)

    wcat1 = jnp.concatenate([W1l.T, W1r.T], axis=1)
    xlr = _project(x, wcat1)
    xl = xlr[:, :H]
    xr = xlr[:, H:]

    seg1 = _make_seg_pass(with_count=True)
    agg1, cnt = seg1(xl, src2d, dst2d, z16, z1, ones)

    w2cat = jnp.concatenate([W2l.T, W2r.T], axis=1)
    hlr, inv = _combine1(agg1[0, :N], agg1[1, :N],
                         cnt[0, :N, None], cnt[1, :N, None],
                         xr, b1l[None, :], w2cat)
    h1l = hlr[:, :H]
    h1r = hlr[:, H:]

    seg2 = _make_seg_pass(with_count=False)
    (agg2,) = seg2(h1l, src2d, dst2d, z16, z1, ones)

    h2 = _combine2(agg2[0, :N], agg2[1, :N], inv, h1r, b2l[None, :])

    dec = _make_decode()
    scores = dec(h2, ia2d, ib2d)
    return scores[:Q]


# speedup vs baseline: 14.3161x; 1.4085x over previous
import functools

import jax
import jax.numpy as jnp
from jax import lax
from jax.experimental import pallas as pl
from jax.experimental.pallas import tpu as pltpu
from jax.experimental.pallas import tpu_sc as plsc

NC, NS, L = 2, 16, 16
NW = NC * NS

N = 10000
E = 320000
D = 128
H = 16
Q = 100000

CHUNK = 128
E_PAD = 327680
E_CHUNKS_PER_TILE = E_PAD // (NW * CHUNK)
NPAD = 10240
ROWS_PER_TILE = NPAD // NS
Q_PAD = 102400
Q_CHUNKS_PER_TILE = Q_PAD // (NW * CHUNK)
Q_PER_TILE = Q_PAD // NW

_f32 = jnp.float32
_i32 = jnp.int32


def _mesh():
    return plsc.VectorSubcoreMesh(
        core_axis_name="c", subcore_axis_name="s",
        num_cores=NC, num_subcores=NS)


NBUF = 4


def _make_seg_pass(with_count):
    out_type = [jax.ShapeDtypeStruct((NC, NPAD, H), _f32)]
    scratch = [
        pltpu.VMEM_SHARED((NPAD, H), _f32),
        pltpu.VMEM((E_CHUNKS_PER_TILE, CHUNK), _i32),
        pltpu.VMEM((E_CHUNKS_PER_TILE, CHUNK), _i32),
        [pltpu.VMEM((CHUNK, H), _f32) for _ in range(NBUF)],
        [pltpu.SemaphoreType.DMA for _ in range(NBUF)],
        [pltpu.SemaphoreType.DMA for _ in range(NBUF)],
    ]
    if with_count:
        out_type.append(jax.ShapeDtypeStruct((NC, NPAD), _f32))
        scratch += [
            pltpu.VMEM_SHARED((NPAD,), _f32),
            pltpu.VMEM((CHUNK,), _f32),
        ]

    def body(table, src2d, dst2d, z16, z1, ones, *rest):
        if with_count:
            (agg_out, cnt_out, agg_sh, src_v, dst_v, rows_v, gsem, ssem,
             cnt_sh, ones_v) = rest
        else:
            agg_out, agg_sh, src_v, dst_v, rows_v, gsem, ssem = rest
        c = lax.axis_index("c")
        s = lax.axis_index("s")
        t = c * NS + s
        nck = E_CHUNKS_PER_TILE
        pltpu.sync_copy(src2d.at[pl.ds(t * nck, nck)], src_v)
        pltpu.sync_copy(dst2d.at[pl.ds(t * nck, nck)], dst_v)
        rpt = ROWS_PER_TILE
        pltpu.sync_copy(z16.at[pl.ds(s * rpt, rpt)], agg_sh.at[pl.ds(s * rpt, rpt)])
        if with_count:
            pltpu.sync_copy(ones, ones_v)
            pltpu.sync_copy(z1.at[pl.ds(s * rpt, rpt)], cnt_sh.at[pl.ds(s * rpt, rpt)])
        plsc.subcore_barrier()

        for b in range(NBUF):
            pltpu.async_copy(table.at[src_v.at[b]], rows_v[b], gsem[b])

        def outer(jo, carry):
            for b in range(NBUF):
                j = jo * NBUF + b
                pltpu.make_async_copy(table.at[src_v.at[j]], rows_v[b],
                                      gsem[b]).wait()
                sc = pltpu.async_copy(rows_v[b], agg_sh.at[dst_v.at[j]],
                                      ssem[b], add=True)
                if with_count:
                    sc1 = pltpu.async_copy(ones_v, cnt_sh.at[dst_v.at[j]],
                                           ssem[b], add=True)
                sc.wait()
                if with_count:
                    sc1.wait()
                jn = lax.rem(j + NBUF, nck)
                pltpu.async_copy(table.at[src_v.at[jn]], rows_v[b], gsem[b])
            return carry

        lax.fori_loop(0, nck // NBUF, outer, 0)
        for b in range(NBUF):
            pltpu.make_async_copy(table.at[src_v.at[b]], rows_v[b],
                                  gsem[b]).wait()
        plsc.subcore_barrier()
        pltpu.sync_copy(agg_sh.at[pl.ds(s * rpt, rpt)],
                        agg_out.at[c, pl.ds(s * rpt, rpt)])
        if with_count:
            pltpu.sync_copy(cnt_sh.at[pl.ds(s * rpt, rpt)],
                            cnt_out.at[c, pl.ds(s * rpt, rpt)])

    return pl.kernel(body, out_type=tuple(out_type), mesh=_mesh(),
                     scratch_types=scratch,
                     compiler_params=pltpu.CompilerParams(
                         use_tc_tiling_on_sc=False))


NBUF_Q = 5


def _decode_body(h2, ia2d, ib2d, out, ia_v, ib_v, ra_v, rb_v, res_v, gsem):
    c = lax.axis_index("c")
    s = lax.axis_index("s")
    t = c * NS + s
    nck = Q_CHUNKS_PER_TILE
    pltpu.sync_copy(ia2d.at[pl.ds(t * nck, nck)], ia_v)
    pltpu.sync_copy(ib2d.at[pl.ds(t * nck, nck)], ib_v)
    lane = lax.iota(_i32, 16)

    for b in range(NBUF_Q):
        pltpu.async_copy(h2.at[ia_v.at[b]], ra_v[b], gsem[b])
        pltpu.async_copy(h2.at[ib_v.at[b]], rb_v[b], gsem[b])

    def outer(jo, carry):
        for b in range(NBUF_Q):
            j = jo * NBUF_Q + b
            pltpu.make_async_copy(h2.at[ia_v.at[j]], ra_v[b], gsem[b]).wait()
            pltpu.make_async_copy(h2.at[ib_v.at[j]], rb_v[b], gsem[b]).wait()
            for g in range(CHUNK // 16):
                ridx = lane + g * 16
                acc = jnp.zeros((16,), _f32)
                for col in range(H):
                    cidx = jnp.full((16,), col, _i32)
                    a = plsc.load_gather(ra_v[b], [ridx, cidx])
                    bb = plsc.load_gather(rb_v[b], [ridx, cidx])
                    acc = acc + a * bb
                sig = 1.0 / (1.0 + jnp.exp(-acc))
                res_v[pl.ds(j * CHUNK + g * 16, 16)] = sig
            jn = lax.rem(j + NBUF_Q, nck)
            pltpu.async_copy(h2.at[ia_v.at[jn]], ra_v[b], gsem[b])
            pltpu.async_copy(h2.at[ib_v.at[jn]], rb_v[b], gsem[b])
        return carry

    lax.fori_loop(0, nck // NBUF_Q, outer, 0)
    for b in range(NBUF_Q):
        pltpu.make_async_copy(h2.at[ia_v.at[b]], ra_v[b], gsem[b]).wait()
        pltpu.make_async_copy(h2.at[ib_v.at[b]], rb_v[b], gsem[b]).wait()
    pltpu.sync_copy(res_v, out.at[pl.ds(t * Q_PER_TILE, Q_PER_TILE)])


def _make_decode():
    return pl.kernel(
        _decode_body,
        out_type=jax.ShapeDtypeStruct((Q_PAD,), _f32),
        mesh=_mesh(),
        scratch_types=[
            pltpu.VMEM((Q_CHUNKS_PER_TILE, CHUNK), _i32),
            pltpu.VMEM((Q_CHUNKS_PER_TILE, CHUNK), _i32),
            [pltpu.VMEM((CHUNK, H), _f32) for _ in range(NBUF_Q)],
            [pltpu.VMEM((CHUNK, H), _f32) for _ in range(NBUF_Q)],
            pltpu.VMEM((Q_PER_TILE,), _f32),
            [pltpu.SemaphoreType.DMA for _ in range(NBUF_Q)],
        ],
        compiler_params=pltpu.CompilerParams(use_tc_tiling_on_sc=False,
                                             needs_layout_passes=False))


_BN = 1000


def _mm_body(x_ref, w_ref, o_ref):
    o_ref[...] = jnp.dot(x_ref[...], w_ref[...], preferred_element_type=_f32)


def _project(x, wcat):
    k = x.shape[1]
    m = wcat.shape[1]
    return pl.pallas_call(
        _mm_body,
        grid=(N // _BN,),
        in_specs=[pl.BlockSpec((_BN, k), lambda i: (i, 0)),
                  pl.BlockSpec((k, m), lambda i: (0, 0))],
        out_specs=pl.BlockSpec((_BN, m), lambda i: (i, 0)),
        out_shape=jax.ShapeDtypeStruct((N, m), _f32),
    )(x, wcat)


def _comb1_body(a0, a1, c0, c1, xr, b, w, o_hlr, o_inv):
    cnt = c0[...] + c1[...]
    inv = 1.0 / jnp.maximum(cnt, 1.0)
    h1 = jnp.maximum((a0[...] + a1[...]) * inv + b[...] + xr[...], 0.0)
    o_hlr[...] = jnp.dot(h1, w[...], preferred_element_type=_f32)
    o_inv[...] = inv


def _combine1(a0, a1, c0, c1, xr, b1l2d, w2cat):
    return pl.pallas_call(
        _comb1_body,
        grid=(N // _BN,),
        in_specs=[pl.BlockSpec((_BN, H), lambda i: (i, 0)),
                  pl.BlockSpec((_BN, H), lambda i: (i, 0)),
                  pl.BlockSpec((_BN, 1), lambda i: (i, 0)),
                  pl.BlockSpec((_BN, 1), lambda i: (i, 0)),
                  pl.BlockSpec((_BN, H), lambda i: (i, 0)),
                  pl.BlockSpec((1, H), lambda i: (0, 0)),
                  pl.BlockSpec((H, 2 * H), lambda i: (0, 0))],
        out_specs=[pl.BlockSpec((_BN, 2 * H), lambda i: (i, 0)),
                   pl.BlockSpec((_BN, 1), lambda i: (i, 0))],
        out_shape=[jax.ShapeDtypeStruct((N, 2 * H), _f32),
                   jax.ShapeDtypeStruct((N, 1), _f32)],
    )(a0, a1, c0, c1, xr, b1l2d, w2cat)


def _comb2_body(a0, a1, inv, hr, b, o):
    o[...] = (a0[...] + a1[...]) * inv[...] + b[...] + hr[...]


def _combine2(a0, a1, inv, h1r, b2l2d):
    return pl.pallas_call(
        _comb2_body,
        grid=(N // _BN,),
        in_specs=[pl.BlockSpec((_BN, H), lambda i: (i, 0)),
                  pl.BlockSpec((_BN, H), lambda i: (i, 0)),
                  pl.BlockSpec((_BN, 1), lambda i: (i, 0)),
                  pl.BlockSpec((_BN, H), lambda i: (i, 0)),
                  pl.BlockSpec((1, H), lambda i: (0, 0))],
        out_specs=pl.BlockSpec((_BN, H), lambda i: (i, 0)),
        out_shape=jax.ShapeDtypeStruct((N, H), _f32),
    )(a0, a1, inv, h1r, b2l2d)


def kernel(x, edge_index, decode_index, W1l, b1l, W1r, W2l, b2l, W2r):
    src = edge_index[0]
    dst = edge_index[1]
    src2d = jnp.pad(src, (0, E_PAD - E)).reshape(E_PAD // CHUNK, CHUNK)
    dst2d = jnp.pad(dst, (0, E_PAD - E), constant_values=N).reshape(
        E_PAD // CHUNK, CHUNK)
    ia2d = jnp.pad(decode_index[0], (0, Q_PAD - Q)).reshape(Q_PAD // CHUNK, CHUNK)
    ib2d = jnp.pad(decode_index[1], (0, Q_PAD - Q)).reshape(Q_PAD // CHUNK, CHUNK)

    z16 = jnp.zeros((NPAD, H), _f32)
    z1 = jnp.zeros((NPAD,), _f32)
    ones = jnp.ones((CHUNK,), _f32)

    wcat1 = jnp.concatenate([W1l.T, W1r.T], axis=1)
    xlr = _project(x, wcat1)
    xl = xlr[:, :H]
    xr = xlr[:, H:]

    seg1 = _make_seg_pass(with_count=True)
    agg1, cnt = seg1(xl, src2d, dst2d, z16, z1, ones)

    w2cat = jnp.concatenate([W2l.T, W2r.T], axis=1)
    hlr, inv = _combine1(agg1[0, :N], agg1[1, :N],
                         cnt[0, :N, None], cnt[1, :N, None],
                         xr, b1l[None, :], w2cat)
    h1l = hlr[:, :H]
    h1r = hlr[:, H:]

    seg2 = _make_seg_pass(with_count=False)
    (agg2,) = seg2(h1l, src2d, dst2d, z16, z1, ones)

    h2 = _combine2(agg2[0, :N], agg2[1, :N], inv, h1r, b2l[None, :])

    dec = _make_decode()
    scores = dec(h2, ia2d, ib2d)
    return scores[:Q]


# speedup vs baseline: 17.7584x; 1.2405x over previous
import functools

import jax
import jax.numpy as jnp
from jax import lax
from jax.experimental import pallas as pl
from jax.experimental.pallas import tpu as pltpu
from jax.experimental.pallas import tpu_sc as plsc

NC, NS, L = 2, 16, 16
NW = NC * NS

N = 10000
E = 320000
D = 128
H = 16
Q = 100000

CHUNK = 128
E_PAD = 327680
E_CHUNKS_PER_TILE = E_PAD // (NW * CHUNK)
NPAD = 10240
ROWS_PER_TILE = NPAD // NS
Q_PAD = 102400
Q_CHUNKS_PER_TILE = Q_PAD // (NW * CHUNK)
Q_PER_TILE = Q_PAD // NW

_f32 = jnp.float32
_i32 = jnp.int32


def _mesh():
    return plsc.VectorSubcoreMesh(
        core_axis_name="c", subcore_axis_name="s",
        num_cores=NC, num_subcores=NS)


NBUF = 4


def _make_seg_pass(with_count):
    out_type = [jax.ShapeDtypeStruct((NC, NPAD, H), _f32)]
    scratch = [
        pltpu.VMEM_SHARED((NPAD, H), _f32),
        pltpu.VMEM_SHARED((NPAD, H), _f32),
        pltpu.VMEM((E_CHUNKS_PER_TILE, CHUNK), _i32),
        pltpu.VMEM((E_CHUNKS_PER_TILE, CHUNK), _i32),
        [pltpu.VMEM((CHUNK, H), _f32) for _ in range(NBUF)],
        [pltpu.SemaphoreType.DMA for _ in range(NBUF)],
        [pltpu.SemaphoreType.DMA for _ in range(NBUF)],
    ]
    if with_count:
        out_type.append(jax.ShapeDtypeStruct((NC, NPAD), _f32))
        scratch += [
            pltpu.VMEM_SHARED((NPAD,), _f32),
            pltpu.VMEM((CHUNK,), _f32),
        ]

    def body(table, src2d, dst2d, z16, z1, ones, *rest):
        if with_count:
            (agg_out, cnt_out, agg_sh, table_sh, src_v, dst_v, rows_v,
             gsem, ssem, cnt_sh, ones_v) = rest
        else:
            agg_out, agg_sh, table_sh, src_v, dst_v, rows_v, gsem, ssem = rest
        c = lax.axis_index("c")
        s = lax.axis_index("s")
        t = c * NS + s
        nck = E_CHUNKS_PER_TILE
        pltpu.sync_copy(src2d.at[pl.ds(t * nck, nck)], src_v)
        pltpu.sync_copy(dst2d.at[pl.ds(t * nck, nck)], dst_v)
        rpt = ROWS_PER_TILE
        pltpu.sync_copy(table.at[pl.ds(s * (N // NS), N // NS)],
                        table_sh.at[pl.ds(s * (N // NS), N // NS)])
        pltpu.sync_copy(z16.at[pl.ds(s * rpt, rpt)], agg_sh.at[pl.ds(s * rpt, rpt)])
        if with_count:
            pltpu.sync_copy(ones, ones_v)
            pltpu.sync_copy(z1.at[pl.ds(s * rpt, rpt)], cnt_sh.at[pl.ds(s * rpt, rpt)])
        plsc.subcore_barrier()

        for b in range(NBUF):
            pltpu.async_copy(table_sh.at[src_v.at[b]], rows_v[b], gsem[b])

        def outer(jo, carry):
            for b in range(NBUF):
                j = jo * NBUF + b
                pltpu.make_async_copy(table_sh.at[src_v.at[j]], rows_v[b],
                                      gsem[b]).wait()
                sc = pltpu.async_copy(rows_v[b], agg_sh.at[dst_v.at[j]],
                                      ssem[b], add=True)
                if with_count:
                    sc1 = pltpu.async_copy(ones_v, cnt_sh.at[dst_v.at[j]],
                                           ssem[b], add=True)
                sc.wait()
                if with_count:
                    sc1.wait()
                jn = lax.rem(j + NBUF, nck)
                pltpu.async_copy(table_sh.at[src_v.at[jn]], rows_v[b], gsem[b])
            return carry

        lax.fori_loop(0, nck // NBUF, outer, 0)
        for b in range(NBUF):
            pltpu.make_async_copy(table_sh.at[src_v.at[b]], rows_v[b],
                                  gsem[b]).wait()
        plsc.subcore_barrier()
        pltpu.sync_copy(agg_sh.at[pl.ds(s * rpt, rpt)],
                        agg_out.at[c, pl.ds(s * rpt, rpt)])
        if with_count:
            pltpu.sync_copy(cnt_sh.at[pl.ds(s * rpt, rpt)],
                            cnt_out.at[c, pl.ds(s * rpt, rpt)])

    return pl.kernel(body, out_type=tuple(out_type), mesh=_mesh(),
                     scratch_types=scratch,
                     compiler_params=pltpu.CompilerParams(
                         use_tc_tiling_on_sc=False))


NBUF_Q = 5


def _decode_body(h2, ia2d, ib2d, out, ia_v, ib_v, ra_v, rb_v, res_v, gsem):
    c = lax.axis_index("c")
    s = lax.axis_index("s")
    t = c * NS + s
    nck = Q_CHUNKS_PER_TILE
    pltpu.sync_copy(ia2d.at[pl.ds(t * nck, nck)], ia_v)
    pltpu.sync_copy(ib2d.at[pl.ds(t * nck, nck)], ib_v)
    lane = lax.iota(_i32, 16)

    for b in range(NBUF_Q):
        pltpu.async_copy(h2.at[ia_v.at[b]], ra_v[b], gsem[b])
        pltpu.async_copy(h2.at[ib_v.at[b]], rb_v[b], gsem[b])

    def outer(jo, carry):
        for b in range(NBUF_Q):
            j = jo * NBUF_Q + b
            pltpu.make_async_copy(h2.at[ia_v.at[j]], ra_v[b], gsem[b]).wait()
            pltpu.make_async_copy(h2.at[ib_v.at[j]], rb_v[b], gsem[b]).wait()
            for g in range(CHUNK // 16):
                ridx = lane + g * 16
                acc = jnp.zeros((16,), _f32)
                for col in range(H):
                    cidx = jnp.full((16,), col, _i32)
                    a = plsc.load_gather(ra_v[b], [ridx, cidx])
                    bb = plsc.load_gather(rb_v[b], [ridx, cidx])
                    acc = acc + a * bb
                sig = 1.0 / (1.0 + jnp.exp(-acc))
                res_v[pl.ds(j * CHUNK + g * 16, 16)] = sig
            jn = lax.rem(j + NBUF_Q, nck)
            pltpu.async_copy(h2.at[ia_v.at[jn]], ra_v[b], gsem[b])
            pltpu.async_copy(h2.at[ib_v.at[jn]], rb_v[b], gsem[b])
        return carry

    lax.fori_loop(0, nck // NBUF_Q, outer, 0)
    for b in range(NBUF_Q):
        pltpu.make_async_copy(h2.at[ia_v.at[b]], ra_v[b], gsem[b]).wait()
        pltpu.make_async_copy(h2.at[ib_v.at[b]], rb_v[b], gsem[b]).wait()
    pltpu.sync_copy(res_v, out.at[pl.ds(t * Q_PER_TILE, Q_PER_TILE)])


def _make_decode():
    return pl.kernel(
        _decode_body,
        out_type=jax.ShapeDtypeStruct((Q_PAD,), _f32),
        mesh=_mesh(),
        scratch_types=[
            pltpu.VMEM((Q_CHUNKS_PER_TILE, CHUNK), _i32),
            pltpu.VMEM((Q_CHUNKS_PER_TILE, CHUNK), _i32),
            [pltpu.VMEM((CHUNK, H), _f32) for _ in range(NBUF_Q)],
            [pltpu.VMEM((CHUNK, H), _f32) for _ in range(NBUF_Q)],
            pltpu.VMEM((Q_PER_TILE,), _f32),
            [pltpu.SemaphoreType.DMA for _ in range(NBUF_Q)],
        ],
        compiler_params=pltpu.CompilerParams(use_tc_tiling_on_sc=False,
                                             needs_layout_passes=False))


_BN = 1000


def _mm_body(x_ref, w_ref, o_ref):
    o_ref[...] = jnp.dot(x_ref[...], w_ref[...], preferred_element_type=_f32)


def _project(x, wcat):
    k = x.shape[1]
    m = wcat.shape[1]
    return pl.pallas_call(
        _mm_body,
        grid=(N // _BN,),
        in_specs=[pl.BlockSpec((_BN, k), lambda i: (i, 0)),
                  pl.BlockSpec((k, m), lambda i: (0, 0))],
        out_specs=pl.BlockSpec((_BN, m), lambda i: (i, 0)),
        out_shape=jax.ShapeDtypeStruct((N, m), _f32),
    )(x, wcat)


def _comb1_body(a0, a1, c0, c1, xr, b, w, o_hlr, o_inv):
    cnt = c0[...] + c1[...]
    inv = 1.0 / jnp.maximum(cnt, 1.0)
    h1 = jnp.maximum((a0[...] + a1[...]) * inv + b[...] + xr[...], 0.0)
    o_hlr[...] = jnp.dot(h1, w[...], preferred_element_type=_f32)
    o_inv[...] = inv


def _combine1(a0, a1, c0, c1, xr, b1l2d, w2cat):
    return pl.pallas_call(
        _comb1_body,
        grid=(N // _BN,),
        in_specs=[pl.BlockSpec((_BN, H), lambda i: (i, 0)),
                  pl.BlockSpec((_BN, H), lambda i: (i, 0)),
                  pl.BlockSpec((_BN, 1), lambda i: (i, 0)),
                  pl.BlockSpec((_BN, 1), lambda i: (i, 0)),
                  pl.BlockSpec((_BN, H), lambda i: (i, 0)),
                  pl.BlockSpec((1, H), lambda i: (0, 0)),
                  pl.BlockSpec((H, 2 * H), lambda i: (0, 0))],
        out_specs=[pl.BlockSpec((_BN, 2 * H), lambda i: (i, 0)),
                   pl.BlockSpec((_BN, 1), lambda i: (i, 0))],
        out_shape=[jax.ShapeDtypeStruct((N, 2 * H), _f32),
                   jax.ShapeDtypeStruct((N, 1), _f32)],
    )(a0, a1, c0, c1, xr, b1l2d, w2cat)


def _comb2_body(a0, a1, inv, hr, b, o):
    o[...] = (a0[...] + a1[...]) * inv[...] + b[...] + hr[...]


def _combine2(a0, a1, inv, h1r, b2l2d):
    return pl.pallas_call(
        _comb2_body,
        grid=(N // _BN,),
        in_specs=[pl.BlockSpec((_BN, H), lambda i: (i, 0)),
                  pl.BlockSpec((_BN, H), lambda i: (i, 0)),
                  pl.BlockSpec((_BN, 1), lambda i: (i, 0)),
                  pl.BlockSpec((_BN, H), lambda i: (i, 0)),
                  pl.BlockSpec((1, H), lambda i: (0, 0))],
        out_specs=pl.BlockSpec((_BN, H), lambda i: (i, 0)),
        out_shape=jax.ShapeDtypeStruct((N, H), _f32),
    )(a0, a1, inv, h1r, b2l2d)


def kernel(x, edge_index, decode_index, W1l, b1l, W1r, W2l, b2l, W2r):
    src = edge_index[0]
    dst = edge_index[1]
    src2d = jnp.pad(src, (0, E_PAD - E)).reshape(E_PAD // CHUNK, CHUNK)
    dst2d = jnp.pad(dst, (0, E_PAD - E), constant_values=N).reshape(
        E_PAD // CHUNK, CHUNK)
    ia2d = jnp.pad(decode_index[0], (0, Q_PAD - Q)).reshape(Q_PAD // CHUNK, CHUNK)
    ib2d = jnp.pad(decode_index[1], (0, Q_PAD - Q)).reshape(Q_PAD // CHUNK, CHUNK)

    z16 = jnp.zeros((NPAD, H), _f32)
    z1 = jnp.zeros((NPAD,), _f32)
    ones = jnp.ones((CHUNK,), _f32)

    wcat1 = jnp.concatenate([W1l.T, W1r.T], axis=1)
    xlr = _project(x, wcat1)
    xl = xlr[:, :H]
    xr = xlr[:, H:]

    seg1 = _make_seg_pass(with_count=True)
    agg1, cnt = seg1(xl, src2d, dst2d, z16, z1, ones)

    w2cat = jnp.concatenate([W2l.T, W2r.T], axis=1)
    hlr, inv = _combine1(agg1[0, :N], agg1[1, :N],
                         cnt[0, :N, None], cnt[1, :N, None],
                         xr, b1l[None, :], w2cat)
    h1l = hlr[:, :H]
    h1r = hlr[:, H:]

    seg2 = _make_seg_pass(with_count=False)
    (agg2,) = seg2(h1l, src2d, dst2d, z16, z1, ones)

    h2 = _combine2(agg2[0, :N], agg2[1, :N], inv, h1r, b2l[None, :])

    dec = _make_decode()
    scores = dec(h2, ia2d, ib2d)
    return scores[:Q]


# speedup vs baseline: 18.3254x; 1.0319x over previous
import functools

import jax
import jax.numpy as jnp
from jax import lax
from jax.experimental import pallas as pl
from jax.experimental.pallas import tpu as pltpu
from jax.experimental.pallas import tpu_sc as plsc

NC, NS, L = 2, 16, 16
NW = NC * NS

N = 10000
E = 320000
D = 128
H = 16
Q = 100000

CHUNK = 128
E_PAD = 327680
E_CHUNKS_PER_TILE = E_PAD // (NW * CHUNK)
NPAD = 10240
ROWS_PER_TILE = NPAD // NS
Q_PAD = 102400
Q_CHUNKS_PER_TILE = Q_PAD // (NW * CHUNK)
Q_PER_TILE = Q_PAD // NW

_f32 = jnp.float32
_i32 = jnp.int32


def _mesh():
    return plsc.VectorSubcoreMesh(
        core_axis_name="c", subcore_axis_name="s",
        num_cores=NC, num_subcores=NS)


NBUF = 4


def _make_seg_pass(with_count):
    out_type = [jax.ShapeDtypeStruct((NC, NPAD, H), _f32)]
    scratch = [
        pltpu.VMEM_SHARED((NPAD, H), _f32),
        pltpu.VMEM_SHARED((NPAD, H), _f32),
        pltpu.VMEM((E_CHUNKS_PER_TILE, CHUNK), _i32),
        pltpu.VMEM((E_CHUNKS_PER_TILE, CHUNK), _i32),
        [pltpu.VMEM((CHUNK, H), _f32) for _ in range(NBUF)],
        [pltpu.SemaphoreType.DMA for _ in range(NBUF)],
        [pltpu.SemaphoreType.DMA for _ in range(NBUF)],
    ]
    if with_count:
        out_type.append(jax.ShapeDtypeStruct((NC, NPAD), _f32))
        scratch += [
            pltpu.VMEM_SHARED((NPAD,), _f32),
            pltpu.VMEM((CHUNK,), _f32),
        ]

    def body(table, src2d, dst2d, z16, z1, ones, *rest):
        if with_count:
            (agg_out, cnt_out, agg_sh, table_sh, src_v, dst_v, rows_v,
             gsem, ssem, cnt_sh, ones_v) = rest
        else:
            agg_out, agg_sh, table_sh, src_v, dst_v, rows_v, gsem, ssem = rest
        c = lax.axis_index("c")
        s = lax.axis_index("s")
        t = c * NS + s
        nck = E_CHUNKS_PER_TILE
        pltpu.sync_copy(src2d.at[pl.ds(t * nck, nck)], src_v)
        pltpu.sync_copy(dst2d.at[pl.ds(t * nck, nck)], dst_v)
        rpt = ROWS_PER_TILE
        pltpu.sync_copy(table.at[pl.ds(s * (N // NS), N // NS)],
                        table_sh.at[pl.ds(s * (N // NS), N // NS)])
        pltpu.sync_copy(z16.at[pl.ds(s * rpt, rpt)], agg_sh.at[pl.ds(s * rpt, rpt)])
        if with_count:
            pltpu.sync_copy(ones, ones_v)
            pltpu.sync_copy(z1.at[pl.ds(s * rpt, rpt)], cnt_sh.at[pl.ds(s * rpt, rpt)])
        plsc.subcore_barrier()

        for b in range(NBUF):
            pltpu.async_copy(table_sh.at[src_v.at[b]], rows_v[b], gsem[b])

        def outer(jo, carry):
            for b in range(NBUF):
                j = jo * NBUF + b
                pltpu.make_async_copy(table_sh.at[src_v.at[j]], rows_v[b],
                                      gsem[b]).wait()
                sc = pltpu.async_copy(rows_v[b], agg_sh.at[dst_v.at[j]],
                                      ssem[b], add=True)
                if with_count:
                    sc1 = pltpu.async_copy(ones_v, cnt_sh.at[dst_v.at[j]],
                                           ssem[b], add=True)
                sc.wait()
                if with_count:
                    sc1.wait()
                jn = lax.rem(j + NBUF, nck)
                pltpu.async_copy(table_sh.at[src_v.at[jn]], rows_v[b], gsem[b])
            return carry

        lax.fori_loop(0, nck // NBUF, outer, 0)
        for b in range(NBUF):
            pltpu.make_async_copy(table_sh.at[src_v.at[b]], rows_v[b],
                                  gsem[b]).wait()
        plsc.subcore_barrier()
        pltpu.sync_copy(agg_sh.at[pl.ds(s * rpt, rpt)],
                        agg_out.at[c, pl.ds(s * rpt, rpt)])
        if with_count:
            pltpu.sync_copy(cnt_sh.at[pl.ds(s * rpt, rpt)],
                            cnt_out.at[c, pl.ds(s * rpt, rpt)])

    return pl.kernel(body, out_type=tuple(out_type), mesh=_mesh(),
                     scratch_types=scratch,
                     compiler_params=pltpu.CompilerParams(
                         use_tc_tiling_on_sc=False))


NBUF_Q = 5


def _decode_body(h2, ia2d, ib2d, out, h2_sh, ia_v, ib_v, ra_v, rb_v, res_v,
                 gsem):
    c = lax.axis_index("c")
    s = lax.axis_index("s")
    t = c * NS + s
    nck = Q_CHUNKS_PER_TILE
    pltpu.sync_copy(ia2d.at[pl.ds(t * nck, nck)], ia_v)
    pltpu.sync_copy(ib2d.at[pl.ds(t * nck, nck)], ib_v)
    pltpu.sync_copy(h2.at[pl.ds(s * (N // NS), N // NS)],
                    h2_sh.at[pl.ds(s * (N // NS), N // NS)])
    plsc.subcore_barrier()
    lane = lax.iota(_i32, 16)

    for b in range(NBUF_Q):
        pltpu.async_copy(h2_sh.at[ia_v.at[b]], ra_v[b], gsem[b])
        pltpu.async_copy(h2_sh.at[ib_v.at[b]], rb_v[b], gsem[b])

    def outer(jo, carry):
        for b in range(NBUF_Q):
            j = jo * NBUF_Q + b
            pltpu.make_async_copy(h2_sh.at[ia_v.at[j]], ra_v[b], gsem[b]).wait()
            pltpu.make_async_copy(h2_sh.at[ib_v.at[j]], rb_v[b], gsem[b]).wait()
            for g in range(CHUNK // 16):
                ridx = lane + g * 16
                acc = jnp.zeros((16,), _f32)
                for col in range(H):
                    cidx = jnp.full((16,), col, _i32)
                    a = plsc.load_gather(ra_v[b], [ridx, cidx])
                    bb = plsc.load_gather(rb_v[b], [ridx, cidx])
                    acc = acc + a * bb
                sig = 1.0 / (1.0 + jnp.exp(-acc))
                res_v[pl.ds(j * CHUNK + g * 16, 16)] = sig
            jn = lax.rem(j + NBUF_Q, nck)
            pltpu.async_copy(h2_sh.at[ia_v.at[jn]], ra_v[b], gsem[b])
            pltpu.async_copy(h2_sh.at[ib_v.at[jn]], rb_v[b], gsem[b])
        return carry

    lax.fori_loop(0, nck // NBUF_Q, outer, 0)
    for b in range(NBUF_Q):
        pltpu.make_async_copy(h2_sh.at[ia_v.at[b]], ra_v[b], gsem[b]).wait()
        pltpu.make_async_copy(h2_sh.at[ib_v.at[b]], rb_v[b], gsem[b]).wait()
    pltpu.sync_copy(res_v, out.at[pl.ds(t * Q_PER_TILE, Q_PER_TILE)])


def _make_decode():
    return pl.kernel(
        _decode_body,
        out_type=jax.ShapeDtypeStruct((Q_PAD,), _f32),
        mesh=_mesh(),
        scratch_types=[
            pltpu.VMEM_SHARED((N, H), _f32),
            pltpu.VMEM((Q_CHUNKS_PER_TILE, CHUNK), _i32),
            pltpu.VMEM((Q_CHUNKS_PER_TILE, CHUNK), _i32),
            [pltpu.VMEM((CHUNK, H), _f32) for _ in range(NBUF_Q)],
            [pltpu.VMEM((CHUNK, H), _f32) for _ in range(NBUF_Q)],
            pltpu.VMEM((Q_PER_TILE,), _f32),
            [pltpu.SemaphoreType.DMA for _ in range(NBUF_Q)],
        ],
        compiler_params=pltpu.CompilerParams(use_tc_tiling_on_sc=False,
                                             needs_layout_passes=False))


_BN = 1000


def _mm_body(x_ref, w_ref, o_ref):
    o_ref[...] = jnp.dot(x_ref[...], w_ref[...], preferred_element_type=_f32)


def _project(x, wcat):
    k = x.shape[1]
    m = wcat.shape[1]
    return pl.pallas_call(
        _mm_body,
        grid=(N // _BN,),
        in_specs=[pl.BlockSpec((_BN, k), lambda i: (i, 0)),
                  pl.BlockSpec((k, m), lambda i: (0, 0))],
        out_specs=pl.BlockSpec((_BN, m), lambda i: (i, 0)),
        out_shape=jax.ShapeDtypeStruct((N, m), _f32),
    )(x, wcat)


def _comb1_body(a0, a1, c0, c1, xr, b, w, o_hlr, o_inv):
    cnt = c0[...] + c1[...]
    inv = 1.0 / jnp.maximum(cnt, 1.0)
    h1 = jnp.maximum((a0[...] + a1[...]) * inv + b[...] + xr[...], 0.0)
    o_hlr[...] = jnp.dot(h1, w[...], preferred_element_type=_f32)
    o_inv[...] = inv


def _combine1(a0, a1, c0, c1, xr, b1l2d, w2cat):
    return pl.pallas_call(
        _comb1_body,
        grid=(N // _BN,),
        in_specs=[pl.BlockSpec((_BN, H), lambda i: (i, 0)),
                  pl.BlockSpec((_BN, H), lambda i: (i, 0)),
                  pl.BlockSpec((_BN, 1), lambda i: (i, 0)),
                  pl.BlockSpec((_BN, 1), lambda i: (i, 0)),
                  pl.BlockSpec((_BN, H), lambda i: (i, 0)),
                  pl.BlockSpec((1, H), lambda i: (0, 0)),
                  pl.BlockSpec((H, 2 * H), lambda i: (0, 0))],
        out_specs=[pl.BlockSpec((_BN, 2 * H), lambda i: (i, 0)),
                   pl.BlockSpec((_BN, 1), lambda i: (i, 0))],
        out_shape=[jax.ShapeDtypeStruct((N, 2 * H), _f32),
                   jax.ShapeDtypeStruct((N, 1), _f32)],
    )(a0, a1, c0, c1, xr, b1l2d, w2cat)


def _comb2_body(a0, a1, inv, hr, b, o):
    o[...] = (a0[...] + a1[...]) * inv[...] + b[...] + hr[...]


def _combine2(a0, a1, inv, h1r, b2l2d):
    return pl.pallas_call(
        _comb2_body,
        grid=(N // _BN,),
        in_specs=[pl.BlockSpec((_BN, H), lambda i: (i, 0)),
                  pl.BlockSpec((_BN, H), lambda i: (i, 0)),
                  pl.BlockSpec((_BN, 1), lambda i: (i, 0)),
                  pl.BlockSpec((_BN, H), lambda i: (i, 0)),
                  pl.BlockSpec((1, H), lambda i: (0, 0))],
        out_specs=pl.BlockSpec((_BN, H), lambda i: (i, 0)),
        out_shape=jax.ShapeDtypeStruct((N, H), _f32),
    )(a0, a1, inv, h1r, b2l2d)


def kernel(x, edge_index, decode_index, W1l, b1l, W1r, W2l, b2l, W2r):
    src = edge_index[0]
    dst = edge_index[1]
    src2d = jnp.pad(src, (0, E_PAD - E)).reshape(E_PAD // CHUNK, CHUNK)
    dst2d = jnp.pad(dst, (0, E_PAD - E), constant_values=N).reshape(
        E_PAD // CHUNK, CHUNK)
    ia2d = jnp.pad(decode_index[0], (0, Q_PAD - Q)).reshape(Q_PAD // CHUNK, CHUNK)
    ib2d = jnp.pad(decode_index[1], (0, Q_PAD - Q)).reshape(Q_PAD // CHUNK, CHUNK)

    z16 = jnp.zeros((NPAD, H), _f32)
    z1 = jnp.zeros((NPAD,), _f32)
    ones = jnp.ones((CHUNK,), _f32)

    wcat1 = jnp.concatenate([W1l.T, W1r.T], axis=1)
    xlr = _project(x, wcat1)
    xl = xlr[:, :H]
    xr = xlr[:, H:]

    seg1 = _make_seg_pass(with_count=True)
    agg1, cnt = seg1(xl, src2d, dst2d, z16, z1, ones)

    w2cat = jnp.concatenate([W2l.T, W2r.T], axis=1)
    hlr, inv = _combine1(agg1[0, :N], agg1[1, :N],
                         cnt[0, :N, None], cnt[1, :N, None],
                         xr, b1l[None, :], w2cat)
    h1l = hlr[:, :H]
    h1r = hlr[:, H:]

    seg2 = _make_seg_pass(with_count=False)
    (agg2,) = seg2(h1l, src2d, dst2d, z16, z1, ones)

    h2 = _combine2(agg2[0, :N], agg2[1, :N], inv, h1r, b2l[None, :])

    dec = _make_decode()
    scores = dec(h2, ia2d, ib2d)
    return scores[:Q]


# speedup vs baseline: 20.2615x; 1.1057x over previous
import functools

import jax
import jax.numpy as jnp
from jax import lax
from jax.experimental import pallas as pl
from jax.experimental.pallas import tpu as pltpu
from jax.experimental.pallas import tpu_sc as plsc

NC, NS, L = 2, 16, 16
NW = NC * NS

N = 10000
E = 320000
D = 128
H = 16
Q = 100000

CHUNK = 125
E_CHUNKS_PER_TILE = E // (NW * CHUNK)
NPAD = 10240
ROWS_PER_TILE = NPAD // NS
TBL_PER_TILE = N // NS
Q_CHUNKS_PER_TILE = Q // (NW * CHUNK)
Q_PER_TILE = Q // NW

NBUF = 4
NBUF_Q = 5

_f32 = jnp.float32
_i32 = jnp.int32


def _mesh():
    return plsc.VectorSubcoreMesh(
        core_axis_name="c", subcore_axis_name="s",
        num_cores=NC, num_subcores=NS)


def _make_seg_pass(with_count):
    out_type = [jax.ShapeDtypeStruct((NC, NPAD, H), _f32)]
    scratch = [
        pltpu.VMEM_SHARED((NPAD, H), _f32),
        pltpu.VMEM_SHARED((N, H), _f32),
        pltpu.VMEM((E_CHUNKS_PER_TILE, CHUNK), _i32),
        pltpu.VMEM((E_CHUNKS_PER_TILE, CHUNK), _i32),
        [pltpu.VMEM((CHUNK, H), _f32) for _ in range(NBUF)],
        [pltpu.SemaphoreType.DMA for _ in range(NBUF)],
        [pltpu.SemaphoreType.DMA for _ in range(NBUF)],
    ]
    if with_count:
        out_type.append(jax.ShapeDtypeStruct((NC, NPAD), _f32))
        scratch += [
            pltpu.VMEM_SHARED((NPAD,), _f32),
            pltpu.VMEM((CHUNK,), _f32),
        ]

    def body(table, src2d, dst2d, z16, z1, ones, *rest):
        if with_count:
            (agg_out, cnt_out, agg_sh, table_sh, src_v, dst_v, rows_v,
             gsem, ssem, cnt_sh, ones_v) = rest
        else:
            agg_out, agg_sh, table_sh, src_v, dst_v, rows_v, gsem, ssem = rest
        c = lax.axis_index("c")
        s = lax.axis_index("s")
        t = c * NS + s
        nck = E_CHUNKS_PER_TILE
        pltpu.sync_copy(src2d.at[pl.ds(t * nck, nck)], src_v)
        pltpu.sync_copy(dst2d.at[pl.ds(t * nck, nck)], dst_v)
        rpt = ROWS_PER_TILE
        pltpu.sync_copy(table.at[pl.ds(s * TBL_PER_TILE, TBL_PER_TILE)],
                        table_sh.at[pl.ds(s * TBL_PER_TILE, TBL_PER_TILE)])
        pltpu.sync_copy(z16.at[pl.ds(s * rpt, rpt)], agg_sh.at[pl.ds(s * rpt, rpt)])
        if with_count:
            pltpu.sync_copy(ones, ones_v)
            pltpu.sync_copy(z1.at[pl.ds(s * rpt, rpt)], cnt_sh.at[pl.ds(s * rpt, rpt)])
        plsc.subcore_barrier()

        for b in range(NBUF):
            pltpu.async_copy(table_sh.at[src_v.at[b]], rows_v[b], gsem[b])

        def outer(jo, carry):
            for b in range(NBUF):
                j = jo * NBUF + b
                pltpu.make_async_copy(table_sh.at[src_v.at[j]], rows_v[b],
                                      gsem[b]).wait()
                sc = pltpu.async_copy(rows_v[b], agg_sh.at[dst_v.at[j]],
                                      ssem[b], add=True)
                if with_count:
                    sc1 = pltpu.async_copy(ones_v, cnt_sh.at[dst_v.at[j]],
                                           ssem[b], add=True)
                sc.wait()
                if with_count:
                    sc1.wait()
                jn = lax.rem(j + NBUF, nck)
                pltpu.async_copy(table_sh.at[src_v.at[jn]], rows_v[b], gsem[b])
            return carry

        lax.fori_loop(0, nck // NBUF, outer, 0)
        for b in range(NBUF):
            pltpu.make_async_copy(table_sh.at[src_v.at[b]], rows_v[b],
                                  gsem[b]).wait()
        plsc.subcore_barrier()
        pltpu.sync_copy(agg_sh.at[pl.ds(s * rpt, rpt)],
                        agg_out.at[c, pl.ds(s * rpt, rpt)])
        if with_count:
            pltpu.sync_copy(cnt_sh.at[pl.ds(s * rpt, rpt)],
                            cnt_out.at[c, pl.ds(s * rpt, rpt)])

    return pl.kernel(body, out_type=tuple(out_type), mesh=_mesh(),
                     scratch_types=scratch,
                     compiler_params=pltpu.CompilerParams(
                         use_tc_tiling_on_sc=False))


def _decode_body(h2, ia2d, ib2d, out, h2_sh, ia_v, ib_v, ra_v, rb_v, res_v,
                 gsem):
    c = lax.axis_index("c")
    s = lax.axis_index("s")
    t = c * NS + s
    nck = Q_CHUNKS_PER_TILE
    pltpu.sync_copy(ia2d.at[pl.ds(t * nck, nck)], ia_v)
    pltpu.sync_copy(ib2d.at[pl.ds(t * nck, nck)], ib_v)
    pltpu.sync_copy(h2.at[pl.ds(s * TBL_PER_TILE, TBL_PER_TILE)],
                    h2_sh.at[pl.ds(s * TBL_PER_TILE, TBL_PER_TILE)])
    plsc.subcore_barrier()
    lane = lax.iota(_i32, 16)

    def ga(j, b):
        return pltpu.make_async_copy(h2_sh.at[ia_v.at[j]],
                                     ra_v[b].at[pl.ds(0, CHUNK)], gsem[b])

    def gb(j, b):
        return pltpu.make_async_copy(h2_sh.at[ib_v.at[j]],
                                     rb_v[b].at[pl.ds(0, CHUNK)], gsem[b])

    for b in range(NBUF_Q):
        ga(b, b).start()
        gb(b, b).start()

    def outer(jo, carry):
        for b in range(NBUF_Q):
            j = jo * NBUF_Q + b
            ga(j, b).wait()
            gb(j, b).wait()
            for g in range(8):
                ridx = lane + g * 16
                acc = jnp.zeros((16,), _f32)
                for col in range(H):
                    cidx = jnp.full((16,), col, _i32)
                    a = plsc.load_gather(ra_v[b], [ridx, cidx])
                    bb = plsc.load_gather(rb_v[b], [ridx, cidx])
                    acc = acc + a * bb
                sig = 1.0 / (1.0 + jnp.exp(-acc))
                res_v[pl.ds(j * CHUNK + g * 16, 16)] = sig
            jn = lax.rem(j + NBUF_Q, nck)
            ga(jn, b).start()
            gb(jn, b).start()
        return carry

    lax.fori_loop(0, nck // NBUF_Q, outer, 0)
    for b in range(NBUF_Q):
        ga(b, b).wait()
        gb(b, b).wait()
    pltpu.sync_copy(res_v.at[pl.ds(0, Q_PER_TILE)], out.at[t])


def _make_decode():
    return pl.kernel(
        _decode_body,
        out_type=jax.ShapeDtypeStruct((NW, Q_PER_TILE), _f32),
        mesh=_mesh(),
        scratch_types=[
            pltpu.VMEM_SHARED((N, H), _f32),
            pltpu.VMEM((Q_CHUNKS_PER_TILE, CHUNK), _i32),
            pltpu.VMEM((Q_CHUNKS_PER_TILE, CHUNK), _i32),
            [pltpu.VMEM((128, H), _f32) for _ in range(NBUF_Q)],
            [pltpu.VMEM((128, H), _f32) for _ in range(NBUF_Q)],
            pltpu.VMEM((Q_PER_TILE + 11,), _f32),
            [pltpu.SemaphoreType.DMA for _ in range(NBUF_Q)],
        ],
        compiler_params=pltpu.CompilerParams(use_tc_tiling_on_sc=False,
                                             needs_layout_passes=False))


_BN = 1024
_GRID = 10


def _proj_body(x_ref, w_ref, ol_ref, or_ref):
    y = jnp.dot(x_ref[...], w_ref[...], preferred_element_type=_f32)
    ol_ref[...] = y[:, :H]
    or_ref[...] = y[:, H:]


def _project(x, wcat):
    return pl.pallas_call(
        _proj_body,
        grid=(_GRID,),
        in_specs=[pl.BlockSpec((_BN, D), lambda i: (i, 0)),
                  pl.BlockSpec((D, 2 * H), lambda i: (0, 0))],
        out_specs=[pl.BlockSpec((_BN, H), lambda i: (i, 0)),
                   pl.BlockSpec((_BN, H), lambda i: (i, 0))],
        out_shape=[jax.ShapeDtypeStruct((N, H), _f32),
                   jax.ShapeDtypeStruct((N, H), _f32)],
    )(x, wcat)


def _comb1_body(a0, a1, inv, xr, b, w, o_hl, o_hr):
    h1 = jnp.maximum((a0[0] + a1[0]) * inv[...] + b[...] + xr[...], 0.0)
    y = jnp.dot(h1, w[...], preferred_element_type=_f32)
    o_hl[...] = y[:, :H]
    o_hr[...] = y[:, H:]


def _combine1(agg1, inv, xr, b1l2d, w2cat):
    return pl.pallas_call(
        _comb1_body,
        grid=(_GRID,),
        in_specs=[pl.BlockSpec((1, _BN, H), lambda i: (0, i, 0)),
                  pl.BlockSpec((1, _BN, H), lambda i: (1, i, 0)),
                  pl.BlockSpec((_BN, 1), lambda i: (i, 0)),
                  pl.BlockSpec((_BN, H), lambda i: (i, 0)),
                  pl.BlockSpec((1, H), lambda i: (0, 0)),
                  pl.BlockSpec((H, 2 * H), lambda i: (0, 0))],
        out_specs=[pl.BlockSpec((_BN, H), lambda i: (i, 0)),
                   pl.BlockSpec((_BN, H), lambda i: (i, 0))],
        out_shape=[jax.ShapeDtypeStruct((NPAD, H), _f32),
                   jax.ShapeDtypeStruct((NPAD, H), _f32)],
    )(agg1, agg1, inv, xr, b1l2d, w2cat)


def _comb2_body(a0, a1, inv, hr, b, o):
    o[...] = (a0[0] + a1[0]) * inv[...] + b[...] + hr[...]


def _combine2(agg2, inv, h1r, b2l2d):
    return pl.pallas_call(
        _comb2_body,
        grid=(_GRID,),
        in_specs=[pl.BlockSpec((1, _BN, H), lambda i: (0, i, 0)),
                  pl.BlockSpec((1, _BN, H), lambda i: (1, i, 0)),
                  pl.BlockSpec((_BN, 1), lambda i: (i, 0)),
                  pl.BlockSpec((_BN, H), lambda i: (i, 0)),
                  pl.BlockSpec((1, H), lambda i: (0, 0))],
        out_specs=pl.BlockSpec((_BN, H), lambda i: (i, 0)),
        out_shape=jax.ShapeDtypeStruct((NPAD, H), _f32),
    )(agg2, agg2, inv, h1r, b2l2d)


def kernel(x, edge_index, decode_index, W1l, b1l, W1r, W2l, b2l, W2r):
    e2d = edge_index.reshape(2, E // CHUNK, CHUNK)
    src2d = e2d[0]
    dst2d = e2d[1]
    q2d = decode_index.reshape(2, Q // CHUNK, CHUNK)
    ia2d = q2d[0]
    ib2d = q2d[1]

    z16 = jnp.zeros((NPAD, H), _f32)
    z1 = jnp.zeros((NPAD,), _f32)
    ones = jnp.ones((CHUNK,), _f32)

    wcat1 = jnp.concatenate([W1l.T, W1r.T], axis=1)
    xl, xr = _project(x, wcat1)

    seg1 = _make_seg_pass(with_count=True)
    agg1, cnt = seg1(xl, src2d, dst2d, z16, z1, ones)

    inv = (1.0 / jnp.maximum(cnt[0] + cnt[1], 1.0))[:, None]

    w2cat = jnp.concatenate([W2l.T, W2r.T], axis=1)
    h1l, h1r = _combine1(agg1, inv, xr, b1l[None, :], w2cat)

    seg2 = _make_seg_pass(with_count=False)
    (agg2,) = seg2(h1l, src2d, dst2d, z16, z1, ones)

    h2 = _combine2(agg2, inv, h1r, b2l[None, :])

    dec = _make_decode()
    scores = dec(h2, ia2d, ib2d)
    return scores.reshape(Q)


# speedup vs baseline: 22.6273x; 1.1168x over previous
import functools

import jax
import jax.numpy as jnp
from jax import lax
from jax.experimental import pallas as pl
from jax.experimental.pallas import tpu as pltpu
from jax.experimental.pallas import tpu_sc as plsc

NC, NS, L = 2, 16, 16
NW = NC * NS

N = 10000
E = 320000
D = 128
H = 16
Q = 100000

CHUNK = 125
E_CHUNKS_PER_TILE = E // (NW * CHUNK)
NPAD = 10240
ROWS_PER_TILE = NPAD // NS
TBL_PER_TILE = N // NS
Q_CHUNKS_PER_TILE = Q // (NW * CHUNK)
Q_PER_TILE = Q // NW

NBUF = 4
NSLOT = 8
NBUF_Q = 5

_f32 = jnp.float32
_i32 = jnp.int32


def _mesh():
    return plsc.VectorSubcoreMesh(
        core_axis_name="c", subcore_axis_name="s",
        num_cores=NC, num_subcores=NS)


def _make_seg_pass(with_count):
    out_type = [jax.ShapeDtypeStruct((NC, NPAD, H), _f32)]
    scratch = [
        pltpu.VMEM_SHARED((NPAD, H), _f32),
        pltpu.VMEM_SHARED((N, H), _f32),
        pltpu.VMEM((E_CHUNKS_PER_TILE, CHUNK), _i32),
        pltpu.VMEM((E_CHUNKS_PER_TILE, CHUNK), _i32),
        [pltpu.VMEM((CHUNK, H), _f32) for _ in range(NSLOT)],
        [pltpu.SemaphoreType.DMA for _ in range(NSLOT)],
        [pltpu.SemaphoreType.DMA for _ in range(NSLOT)],
    ]
    if with_count:
        out_type.append(jax.ShapeDtypeStruct((NC, NPAD), _f32))
        scratch += [
            pltpu.VMEM_SHARED((NPAD,), _f32),
            pltpu.VMEM((CHUNK,), _f32),
        ]

    def body(table, src2d, dst2d, z16, z1, ones, *rest):
        if with_count:
            (agg_out, cnt_out, agg_sh, table_sh, src_v, dst_v, rows_v,
             gsem, ssem, cnt_sh, ones_v) = rest
        else:
            agg_out, agg_sh, table_sh, src_v, dst_v, rows_v, gsem, ssem = rest
        c = lax.axis_index("c")
        s = lax.axis_index("s")
        t = c * NS + s
        nck = E_CHUNKS_PER_TILE
        pltpu.sync_copy(src2d.at[pl.ds(t * nck, nck)], src_v)
        pltpu.sync_copy(dst2d.at[pl.ds(t * nck, nck)], dst_v)
        rpt = ROWS_PER_TILE
        pltpu.sync_copy(table.at[pl.ds(s * TBL_PER_TILE, TBL_PER_TILE)],
                        table_sh.at[pl.ds(s * TBL_PER_TILE, TBL_PER_TILE)])
        pltpu.sync_copy(z16.at[pl.ds(s * rpt, rpt)], agg_sh.at[pl.ds(s * rpt, rpt)])
        if with_count:
            pltpu.sync_copy(ones, ones_v)
            pltpu.sync_copy(z1.at[pl.ds(s * rpt, rpt)], cnt_sh.at[pl.ds(s * rpt, rpt)])
        plsc.subcore_barrier()

        def gath(j, b):
            return pltpu.make_async_copy(table_sh.at[src_v.at[j]],
                                         rows_v[b], gsem[b])

        def scat(j, b):
            return pltpu.make_async_copy(rows_v[b], agg_sh.at[dst_v.at[j]],
                                         ssem[b])

        def scat_cnt(j, b):
            return pltpu.make_async_copy(ones_v, cnt_sh.at[dst_v.at[j]],
                                         ssem[b])

        def step(j, b, first_lap):
            gath(j, b).wait()
            pltpu.async_copy(rows_v[b], agg_sh.at[dst_v.at[j]], ssem[b],
                             add=True)
            if with_count:
                pltpu.async_copy(ones_v, cnt_sh.at[dst_v.at[j]], ssem[b],
                                 add=True)
            bf = (b + NBUF) % NSLOT
            if not (first_lap and b < NBUF):
                scat(j, bf).wait()
                if with_count:
                    scat_cnt(j, bf).wait()
            jn = lax.rem(j + NBUF, nck)
            gath(jn, bf).start()

        for b in range(NBUF):
            gath(b, b).start()
        for j in range(NSLOT):
            step(j, j, True)

        def outer(jo, carry):
            for b in range(NSLOT):
                step(jo * NSLOT + b, b, False)
            return carry

        lax.fori_loop(1, nck // NSLOT, outer, 0)
        for b in range(NBUF):
            gath(b, b).wait()
        for b in range(NBUF, NSLOT):
            scat(0, b).wait()
            if with_count:
                scat_cnt(0, b).wait()
        plsc.subcore_barrier()
        pltpu.sync_copy(agg_sh.at[pl.ds(s * rpt, rpt)],
                        agg_out.at[c, pl.ds(s * rpt, rpt)])
        if with_count:
            pltpu.sync_copy(cnt_sh.at[pl.ds(s * rpt, rpt)],
                            cnt_out.at[c, pl.ds(s * rpt, rpt)])

    return pl.kernel(body, out_type=tuple(out_type), mesh=_mesh(),
                     scratch_types=scratch,
                     compiler_params=pltpu.CompilerParams(
                         use_tc_tiling_on_sc=False))


def _decode_body(agg2, inv16, hc, ia2d, ib2d, out, h2_sh, a0_v, a1_v, iv_v,
                 hc_v, h2_v, ia_v, ib_v, ra_v, rb_v, res_v, gsem):
    c = lax.axis_index("c")
    s = lax.axis_index("s")
    t = c * NS + s
    nck = Q_CHUNKS_PER_TILE
    pltpu.sync_copy(ia2d.at[pl.ds(t * nck, nck)], ia_v)
    pltpu.sync_copy(ib2d.at[pl.ds(t * nck, nck)], ib_v)
    tp = TBL_PER_TILE
    pltpu.sync_copy(agg2.at[0, pl.ds(s * tp, tp)], a0_v)
    pltpu.sync_copy(agg2.at[1, pl.ds(s * tp, tp)], a1_v)
    pltpu.sync_copy(inv16.at[pl.ds(s * tp, tp)], iv_v)
    pltpu.sync_copy(hc.at[pl.ds(s * tp, tp)], hc_v)

    def h2row(r, carry):
        h2_v[r, :] = (a0_v[r, :] + a1_v[r, :]) * iv_v[r, :] + hc_v[r, :]
        return carry

    lax.fori_loop(0, tp, h2row, 0)
    pltpu.sync_copy(h2_v, h2_sh.at[pl.ds(s * tp, tp)])
    plsc.subcore_barrier()
    lane = lax.iota(_i32, 16)

    def ga(j, b):
        return pltpu.make_async_copy(h2_sh.at[ia_v.at[j]],
                                     ra_v[b].at[pl.ds(0, CHUNK)], gsem[b])

    def gb(j, b):
        return pltpu.make_async_copy(h2_sh.at[ib_v.at[j]],
                                     rb_v[b].at[pl.ds(0, CHUNK)], gsem[b])

    for b in range(NBUF_Q):
        ga(b, b).start()
        gb(b, b).start()

    def outer(jo, carry):
        for b in range(NBUF_Q):
            j = jo * NBUF_Q + b
            ga(j, b).wait()
            gb(j, b).wait()
            for g in range(8):
                ridx = lane + g * 16
                acc = jnp.zeros((16,), _f32)
                for col in range(H):
                    cidx = jnp.full((16,), col, _i32)
                    a = plsc.load_gather(ra_v[b], [ridx, cidx])
                    bb = plsc.load_gather(rb_v[b], [ridx, cidx])
                    acc = acc + a * bb
                sig = 1.0 / (1.0 + jnp.exp(-acc))
                res_v[pl.ds(j * CHUNK + g * 16, 16)] = sig
            jn = lax.rem(j + NBUF_Q, nck)
            ga(jn, b).start()
            gb(jn, b).start()
        return carry

    lax.fori_loop(0, nck // NBUF_Q, outer, 0)
    for b in range(NBUF_Q):
        ga(b, b).wait()
        gb(b, b).wait()
    pltpu.sync_copy(res_v.at[pl.ds(0, Q_PER_TILE)], out.at[t])


def _make_decode():
    return pl.kernel(
        _decode_body,
        out_type=jax.ShapeDtypeStruct((NW, Q_PER_TILE), _f32),
        mesh=_mesh(),
        scratch_types=[
            pltpu.VMEM_SHARED((N, H), _f32),
            pltpu.VMEM((TBL_PER_TILE, H), _f32),
            pltpu.VMEM((TBL_PER_TILE, H), _f32),
            pltpu.VMEM((TBL_PER_TILE, H), _f32),
            pltpu.VMEM((TBL_PER_TILE, H), _f32),
            pltpu.VMEM((TBL_PER_TILE, H), _f32),
            pltpu.VMEM((Q_CHUNKS_PER_TILE, CHUNK), _i32),
            pltpu.VMEM((Q_CHUNKS_PER_TILE, CHUNK), _i32),
            [pltpu.VMEM((128, H), _f32) for _ in range(NBUF_Q)],
            [pltpu.VMEM((128, H), _f32) for _ in range(NBUF_Q)],
            pltpu.VMEM((Q_PER_TILE + 11,), _f32),
            [pltpu.SemaphoreType.DMA for _ in range(NBUF_Q)],
        ],
        compiler_params=pltpu.CompilerParams(use_tc_tiling_on_sc=False,
                                             needs_layout_passes=False))


_BN = 2048
_GRID = 5


def _proj_body(x_ref, w_ref, ol_ref, or_ref):
    y = jnp.dot(x_ref[...], w_ref[...], preferred_element_type=_f32)
    ol_ref[...] = y[:, :H]
    or_ref[...] = y[:, H:]


def _project(x, wcat):
    return pl.pallas_call(
        _proj_body,
        grid=(_GRID,),
        in_specs=[pl.BlockSpec((_BN, D), lambda i: (i, 0)),
                  pl.BlockSpec((D, 2 * H), lambda i: (0, 0))],
        out_specs=[pl.BlockSpec((_BN, H), lambda i: (i, 0)),
                   pl.BlockSpec((_BN, H), lambda i: (i, 0))],
        out_shape=[jax.ShapeDtypeStruct((N, H), _f32),
                   jax.ShapeDtypeStruct((N, H), _f32)],
    )(x, wcat)


def _comb1_body(a0, a1, inv, xr, b1, b2, w, o_hl, o_hc, o_inv):
    iv = inv[...]
    h1 = jnp.maximum((a0[0] + a1[0]) * iv + b1[...] + xr[...], 0.0)
    y = jnp.dot(h1, w[...], preferred_element_type=_f32)
    o_hl[...] = y[:, :H]
    o_hc[...] = y[:, H:] + b2[...]
    o_inv[...] = jnp.broadcast_to(iv, iv.shape[:1] + (H,))


def _combine1(agg1, inv, xr, b1l2d, b2l2d, w2cat):
    return pl.pallas_call(
        _comb1_body,
        grid=(_GRID,),
        in_specs=[pl.BlockSpec((1, _BN, H), lambda i: (0, i, 0)),
                  pl.BlockSpec((1, _BN, H), lambda i: (1, i, 0)),
                  pl.BlockSpec((_BN, 1), lambda i: (i, 0)),
                  pl.BlockSpec((_BN, H), lambda i: (i, 0)),
                  pl.BlockSpec((1, H), lambda i: (0, 0)),
                  pl.BlockSpec((1, H), lambda i: (0, 0)),
                  pl.BlockSpec((H, 2 * H), lambda i: (0, 0))],
        out_specs=[pl.BlockSpec((_BN, H), lambda i: (i, 0)),
                   pl.BlockSpec((_BN, H), lambda i: (i, 0)),
                   pl.BlockSpec((_BN, H), lambda i: (i, 0))],
        out_shape=[jax.ShapeDtypeStruct((NPAD, H), _f32),
                   jax.ShapeDtypeStruct((NPAD, H), _f32),
                   jax.ShapeDtypeStruct((NPAD, H), _f32)],
    )(agg1, agg1, inv, xr, b1l2d, b2l2d, w2cat)


def kernel(x, edge_index, decode_index, W1l, b1l, W1r, W2l, b2l, W2r):
    e2d = edge_index.reshape(2, E // CHUNK, CHUNK)
    src2d = e2d[0]
    dst2d = e2d[1]
    q2d = decode_index.reshape(2, Q // CHUNK, CHUNK)
    ia2d = q2d[0]
    ib2d = q2d[1]

    z16 = jnp.zeros((NPAD, H), _f32)
    z1 = jnp.zeros((NPAD,), _f32)
    ones = jnp.ones((CHUNK,), _f32)

    wcat1 = jnp.concatenate([W1l.T, W1r.T], axis=1)
    xl, xr = _project(x, wcat1)

    seg1 = _make_seg_pass(with_count=True)
    agg1, cnt = seg1(xl, src2d, dst2d, z16, z1, ones)

    inv = (1.0 / jnp.maximum(cnt[0] + cnt[1], 1.0))[:, None]

    w2cat = jnp.concatenate([W2l.T, W2r.T], axis=1)
    h1l, hc, inv16 = _combine1(agg1, inv, xr, b1l[None, :], b2l[None, :],
                               w2cat)

    seg2 = _make_seg_pass(with_count=False)
    (agg2,) = seg2(h1l, src2d, dst2d, z16, z1, ones)

    dec = _make_decode()
    scores = dec(agg2, inv16, hc, ia2d, ib2d)
    return scores.reshape(Q)


# speedup vs baseline: 23.6999x; 1.0474x over previous
import functools

import jax
import jax.numpy as jnp
from jax import lax
from jax.experimental import pallas as pl
from jax.experimental.pallas import tpu as pltpu
from jax.experimental.pallas import tpu_sc as plsc

NC, NS, L = 2, 16, 16
NW = NC * NS

N = 10000
E = 320000
D = 128
H = 16
Q = 100000

CHUNK = 125
E_CHUNKS_PER_TILE = E // (NW * CHUNK)
NPAD = 10240
ROWS_PER_TILE = NPAD // NS
TBL_PER_TILE = N // NS
Q_CHUNKS_PER_TILE = Q // (NW * CHUNK)
Q_PER_TILE = Q // NW

NBUF = 4
NSLOT = 8
NBUF_Q = 5

_f32 = jnp.float32
_i32 = jnp.int32


def _mesh():
    return plsc.VectorSubcoreMesh(
        core_axis_name="c", subcore_axis_name="s",
        num_cores=NC, num_subcores=NS)


def _make_seg_pass(with_count):
    out_type = [jax.ShapeDtypeStruct((NC, NPAD, H), _f32)]
    scratch = [
        pltpu.VMEM_SHARED((NPAD, H), _f32),
        pltpu.VMEM_SHARED((N, H), _f32),
        pltpu.VMEM((E_CHUNKS_PER_TILE, CHUNK), _i32),
        pltpu.VMEM((E_CHUNKS_PER_TILE, CHUNK), _i32),
        [pltpu.VMEM((CHUNK, H), _f32) for _ in range(NSLOT)],
        [pltpu.SemaphoreType.DMA for _ in range(NSLOT)],
        [pltpu.SemaphoreType.DMA for _ in range(NSLOT)],
    ]
    if with_count:
        out_type.append(jax.ShapeDtypeStruct((NC, NPAD), _f32))
        scratch += [
            pltpu.VMEM_SHARED((NPAD,), _f32),
            pltpu.VMEM((CHUNK,), _f32),
        ]

    def body(table, src2d, dst2d, z16, z1, ones, *rest):
        if with_count:
            (agg_out, cnt_out, agg_sh, table_sh, src_v, dst_v, rows_v,
             gsem, ssem, cnt_sh, ones_v) = rest
        else:
            agg_out, agg_sh, table_sh, src_v, dst_v, rows_v, gsem, ssem = rest
        c = lax.axis_index("c")
        s = lax.axis_index("s")
        t = c * NS + s
        nck = E_CHUNKS_PER_TILE
        pltpu.sync_copy(src2d.at[pl.ds(t * nck, nck)], src_v)
        pltpu.sync_copy(dst2d.at[pl.ds(t * nck, nck)], dst_v)
        rpt = ROWS_PER_TILE
        pltpu.sync_copy(table.at[pl.ds(s * TBL_PER_TILE, TBL_PER_TILE)],
                        table_sh.at[pl.ds(s * TBL_PER_TILE, TBL_PER_TILE)])
        pltpu.sync_copy(z16.at[pl.ds(s * rpt, rpt)], agg_sh.at[pl.ds(s * rpt, rpt)])
        if with_count:
            pltpu.sync_copy(ones, ones_v)
            pltpu.sync_copy(z1.at[pl.ds(s * rpt, rpt)], cnt_sh.at[pl.ds(s * rpt, rpt)])
        plsc.subcore_barrier()

        def gath(j, b):
            return pltpu.make_async_copy(table_sh.at[src_v.at[j]],
                                         rows_v[b], gsem[b])

        def scat(j, b):
            return pltpu.make_async_copy(rows_v[b], agg_sh.at[dst_v.at[j]],
                                         ssem[b])

        def scat_cnt(j, b):
            return pltpu.make_async_copy(ones_v, cnt_sh.at[dst_v.at[j]],
                                         ssem[b])

        def step(j, b, first_lap):
            gath(j, b).wait()
            pltpu.async_copy(rows_v[b], agg_sh.at[dst_v.at[j]], ssem[b],
                             add=True)
            if with_count:
                pltpu.async_copy(ones_v, cnt_sh.at[dst_v.at[j]], ssem[b],
                                 add=True)
            bf = (b + NBUF) % NSLOT
            if not (first_lap and b < NBUF):
                scat(j, bf).wait()
                if with_count:
                    scat_cnt(j, bf).wait()
            jn = lax.rem(j + NBUF, nck)
            gath(jn, bf).start()

        for b in range(NBUF):
            gath(b, b).start()
        for j in range(NSLOT):
            step(j, j, True)

        def outer(jo, carry):
            for b in range(NSLOT):
                step(jo * NSLOT + b, b, False)
            return carry

        lax.fori_loop(1, nck // NSLOT, outer, 0)
        for b in range(NBUF):
            gath(b, b).wait()
        for b in range(NBUF, NSLOT):
            scat(0, b).wait()
            if with_count:
                scat_cnt(0, b).wait()
        plsc.subcore_barrier()
        pltpu.sync_copy(agg_sh.at[pl.ds(s * rpt, rpt)],
                        agg_out.at[c, pl.ds(s * rpt, rpt)])
        if with_count:
            pltpu.sync_copy(cnt_sh.at[pl.ds(s * rpt, rpt)],
                            cnt_out.at[c, pl.ds(s * rpt, rpt)])

    return pl.kernel(body, out_type=tuple(out_type), mesh=_mesh(),
                     scratch_types=scratch,
                     compiler_params=pltpu.CompilerParams(
                         use_tc_tiling_on_sc=False))


def _decode_body(agg2, inv16, hc, ia2d, ib2d, out, h2_sh, a0_v, a1_v, iv_v,
                 hc_v, h2_v, ia_v, ib_v, ra_v, rb_v, res_v, gsem):
    c = lax.axis_index("c")
    s = lax.axis_index("s")
    t = c * NS + s
    nck = Q_CHUNKS_PER_TILE
    pltpu.sync_copy(ia2d.at[pl.ds(t * nck, nck)], ia_v)
    pltpu.sync_copy(ib2d.at[pl.ds(t * nck, nck)], ib_v)
    tp = TBL_PER_TILE
    pltpu.sync_copy(agg2.at[0, pl.ds(s * tp, tp)], a0_v)
    pltpu.sync_copy(agg2.at[1, pl.ds(s * tp, tp)], a1_v)
    pltpu.sync_copy(inv16.at[pl.ds(s * tp, tp)], iv_v)
    pltpu.sync_copy(hc.at[pl.ds(s * tp, tp)], hc_v)

    def h2row(r, carry):
        h2_v[r, :] = (a0_v[r, :] + a1_v[r, :]) * iv_v[r, :] + hc_v[r, :]
        return carry

    lax.fori_loop(0, tp, h2row, 0)
    pltpu.sync_copy(h2_v, h2_sh.at[pl.ds(s * tp, tp)])
    plsc.subcore_barrier()
    lane = lax.iota(_i32, 16)
    cidxs = [lax.rem(lane + k, 16) for k in range(H)]

    def ga(j, b):
        return pltpu.make_async_copy(h2_sh.at[ia_v.at[j]],
                                     ra_v[b].at[pl.ds(0, CHUNK)], gsem[b])

    def gb(j, b):
        return pltpu.make_async_copy(h2_sh.at[ib_v.at[j]],
                                     rb_v[b].at[pl.ds(0, CHUNK)], gsem[b])

    for b in range(NBUF_Q):
        ga(b, b).start()
        gb(b, b).start()

    def outer(jo, carry):
        for b in range(NBUF_Q):
            j = jo * NBUF_Q + b
            ga(j, b).wait()
            gb(j, b).wait()
            for g in range(8):
                ridx = lane + g * 16
                acc = jnp.zeros((16,), _f32)
                for col in range(H):
                    a = plsc.load_gather(ra_v[b], [ridx, cidxs[col]])
                    bb = plsc.load_gather(rb_v[b], [ridx, cidxs[col]])
                    acc = acc + a * bb
                sig = 1.0 / (1.0 + jnp.exp(-acc))
                res_v[pl.ds(j * CHUNK + g * 16, 16)] = sig
            jn = lax.rem(j + NBUF_Q, nck)
            ga(jn, b).start()
            gb(jn, b).start()
        return carry

    lax.fori_loop(0, nck // NBUF_Q, outer, 0)
    for b in range(NBUF_Q):
        ga(b, b).wait()
        gb(b, b).wait()
    pltpu.sync_copy(res_v.at[pl.ds(0, Q_PER_TILE)], out.at[t])


def _make_decode():
    return pl.kernel(
        _decode_body,
        out_type=jax.ShapeDtypeStruct((NW, Q_PER_TILE), _f32),
        mesh=_mesh(),
        scratch_types=[
            pltpu.VMEM_SHARED((N, H), _f32),
            pltpu.VMEM((TBL_PER_TILE, H), _f32),
            pltpu.VMEM((TBL_PER_TILE, H), _f32),
            pltpu.VMEM((TBL_PER_TILE, H), _f32),
            pltpu.VMEM((TBL_PER_TILE, H), _f32),
            pltpu.VMEM((TBL_PER_TILE, H), _f32),
            pltpu.VMEM((Q_CHUNKS_PER_TILE, CHUNK), _i32),
            pltpu.VMEM((Q_CHUNKS_PER_TILE, CHUNK), _i32),
            [pltpu.VMEM((128, H), _f32) for _ in range(NBUF_Q)],
            [pltpu.VMEM((128, H), _f32) for _ in range(NBUF_Q)],
            pltpu.VMEM((Q_PER_TILE + 11,), _f32),
            [pltpu.SemaphoreType.DMA for _ in range(NBUF_Q)],
        ],
        compiler_params=pltpu.CompilerParams(use_tc_tiling_on_sc=False,
                                             needs_layout_passes=False))


_BN = 2048
_GRID = 5


def _proj_body(x_ref, w_ref, ol_ref, or_ref):
    y = jnp.dot(x_ref[...], w_ref[...], preferred_element_type=_f32)
    ol_ref[...] = y[:, :H]
    or_ref[...] = y[:, H:]


def _project(x, wcat):
    return pl.pallas_call(
        _proj_body,
        grid=(_GRID,),
        in_specs=[pl.BlockSpec((_BN, D), lambda i: (i, 0)),
                  pl.BlockSpec((D, 2 * H), lambda i: (0, 0))],
        out_specs=[pl.BlockSpec((_BN, H), lambda i: (i, 0)),
                   pl.BlockSpec((_BN, H), lambda i: (i, 0))],
        out_shape=[jax.ShapeDtypeStruct((N, H), _f32),
                   jax.ShapeDtypeStruct((N, H), _f32)],
    )(x, wcat)


def _comb1_body(a0, a1, inv, xr, b1, b2, w, o_hl, o_hc, o_inv):
    iv = inv[...]
    h1 = jnp.maximum((a0[0] + a1[0]) * iv + b1[...] + xr[...], 0.0)
    y = jnp.dot(h1, w[...], preferred_element_type=_f32)
    o_hl[...] = y[:, :H]
    o_hc[...] = y[:, H:] + b2[...]
    o_inv[...] = jnp.broadcast_to(iv, iv.shape[:1] + (H,))


def _combine1(agg1, inv, xr, b1l2d, b2l2d, w2cat):
    return pl.pallas_call(
        _comb1_body,
        grid=(_GRID,),
        in_specs=[pl.BlockSpec((1, _BN, H), lambda i: (0, i, 0)),
                  pl.BlockSpec((1, _BN, H), lambda i: (1, i, 0)),
                  pl.BlockSpec((_BN, 1), lambda i: (i, 0)),
                  pl.BlockSpec((_BN, H), lambda i: (i, 0)),
                  pl.BlockSpec((1, H), lambda i: (0, 0)),
                  pl.BlockSpec((1, H), lambda i: (0, 0)),
                  pl.BlockSpec((H, 2 * H), lambda i: (0, 0))],
        out_specs=[pl.BlockSpec((_BN, H), lambda i: (i, 0)),
                   pl.BlockSpec((_BN, H), lambda i: (i, 0)),
                   pl.BlockSpec((_BN, H), lambda i: (i, 0))],
        out_shape=[jax.ShapeDtypeStruct((NPAD, H), _f32),
                   jax.ShapeDtypeStruct((NPAD, H), _f32),
                   jax.ShapeDtypeStruct((NPAD, H), _f32)],
    )(agg1, agg1, inv, xr, b1l2d, b2l2d, w2cat)


def kernel(x, edge_index, decode_index, W1l, b1l, W1r, W2l, b2l, W2r):
    e2d = edge_index.reshape(2, E // CHUNK, CHUNK)
    src2d = e2d[0]
    dst2d = e2d[1]
    q2d = decode_index.reshape(2, Q // CHUNK, CHUNK)
    ia2d = q2d[0]
    ib2d = q2d[1]

    z16 = jnp.zeros((NPAD, H), _f32)
    z1 = jnp.zeros((NPAD,), _f32)
    ones = jnp.ones((CHUNK,), _f32)

    wcat1 = jnp.concatenate([W1l.T, W1r.T], axis=1)
    xl, xr = _project(x, wcat1)

    seg1 = _make_seg_pass(with_count=True)
    agg1, cnt = seg1(xl, src2d, dst2d, z16, z1, ones)

    inv = (1.0 / jnp.maximum(cnt[0] + cnt[1], 1.0))[:, None]

    w2cat = jnp.concatenate([W2l.T, W2r.T], axis=1)
    h1l, hc, inv16 = _combine1(agg1, inv, xr, b1l[None, :], b2l[None, :],
                               w2cat)

    seg2 = _make_seg_pass(with_count=False)
    (agg2,) = seg2(h1l, src2d, dst2d, z16, z1, ones)

    dec = _make_decode()
    scores = dec(agg2, inv16, hc, ia2d, ib2d)
    return scores.reshape(Q)
